# baseline (speedup 1.0000x reference)
import functools

import jax
import jax.numpy as jnp
from jax import lax
from jax.experimental import pallas as pl
from jax.experimental.pallas import tpu as pltpu
from jax.experimental.pallas import tpu_sc as plsc

CHUNK = 128


@functools.lru_cache(maxsize=None)
def _make(B, D):
    info = plsc.get_sparse_core_info()
    NC, NS = info.num_cores, info.num_subcores
    NW = NC * NS
    assert B % (NW * CHUNK) == 0
    b_per_w = B // NW
    n_chunks = b_per_w // CHUNK
    mesh = plsc.VectorSubcoreMesh(core_axis_name="c", subcore_axis_name="s")

    @functools.partial(
        pl.kernel,
        mesh=mesh,
        out_type=jax.ShapeDtypeStruct((B, D), jnp.float32),
        scratch_types=[
            pltpu.VMEM((n_chunks, CHUNK), jnp.int32),
            pltpu.VMEM((CHUNK, D), jnp.float32),
            pltpu.SemaphoreType.DMA,
        ],
        compiler_params=pltpu.CompilerParams(use_tc_tiling_on_sc=False),
    )
    def k(idx_hbm, table_hbm, out_hbm, idx_v, rows_v, sem):
        wid = lax.axis_index("s") * NC + lax.axis_index("c")
        pltpu.sync_copy(idx_hbm.at[wid], idx_v)
        base = wid * b_per_w

        def body(c, carry):
            pltpu.async_copy(table_hbm.at[idx_v.at[c]], rows_v, sem).wait()
            pltpu.sync_copy(rows_v, out_hbm.at[pl.ds(base + c * CHUNK, CHUNK)])
            return carry

        lax.fori_loop(0, n_chunks, body, 0)

    return k


def kernel(index, table):
    batch, fields = index.shape
    D = table.shape[1]
    B = batch * fields
    info = plsc.get_sparse_core_info()
    NW = info.num_cores * info.num_subcores
    b_per_w = B // NW
    idx = index.reshape(NW, b_per_w // CHUNK, CHUNK).astype(jnp.int32)
    out = _make(B, D)(idx, table)
    return out.reshape(batch, fields, D)


# speedup vs baseline: 1.0724x; 1.0724x over previous
import functools

import jax
import jax.numpy as jnp
from jax import lax
from jax.experimental import pallas as pl
from jax.experimental.pallas import tpu as pltpu
from jax.experimental.pallas import tpu_sc as plsc

CHUNK = 128
NBUF = 4


@functools.lru_cache(maxsize=None)
def _make(B, D):
    info = plsc.get_sparse_core_info()
    NC, NS = info.num_cores, info.num_subcores
    NW = NC * NS
    assert B % (NW * CHUNK * NBUF) == 0
    b_per_w = B // NW
    n_chunks = b_per_w // CHUNK
    n_groups = n_chunks // NBUF
    mesh = plsc.VectorSubcoreMesh(core_axis_name="c", subcore_axis_name="s")

    @functools.partial(
        pl.kernel,
        mesh=mesh,
        out_type=jax.ShapeDtypeStruct((B, D), jnp.float32),
        scratch_types=[
            pltpu.VMEM((n_chunks, CHUNK), jnp.int32),
            pltpu.VMEM((2, NBUF, CHUNK, D), jnp.float32),
            pltpu.SemaphoreType.DMA,
            pltpu.SemaphoreType.DMA,
        ],
        compiler_params=pltpu.CompilerParams(use_tc_tiling_on_sc=False),
    )
    def k(idx_hbm, table_hbm, out_hbm, idx_v, bufs, gsem, wsem):
        wid = lax.axis_index("s") * NC + lax.axis_index("c")
        pltpu.sync_copy(idx_hbm.at[wid], idx_v)
        base = wid * b_per_w

        def issue_gathers(g, p):
            for b in range(NBUF):
                pltpu.async_copy(
                    table_hbm.at[idx_v.at[g * NBUF + b]], bufs.at[p, b], gsem)

        def wait_gathers(p):
            for b in range(NBUF):
                pltpu.make_async_copy(
                    table_hbm.at[idx_v.at[0]], bufs.at[p, b], gsem).wait()

        def issue_writes(g, p):
            for b in range(NBUF):
                c = g * NBUF + b
                pltpu.async_copy(
                    bufs.at[p, b],
                    out_hbm.at[pl.ds(base + c * CHUNK, CHUNK)], wsem)

        def wait_writes(p):
            for b in range(NBUF):
                pltpu.make_async_copy(
                    bufs.at[p, b], out_hbm.at[pl.ds(base, CHUNK)], wsem).wait()

        issue_gathers(0, 0)

        def body(g, carry):
            p = lax.rem(g, 2)
            wait_gathers(p)

            @pl.when(g > 0)
            def _():
                wait_writes(1 - p)

            @pl.when(g + 1 < n_groups)
            def _():
                issue_gathers(g + 1, 1 - p)

            issue_writes(g, p)
            return carry

        lax.fori_loop(0, n_groups, body, 0)
        wait_writes((n_groups - 1) % 2)

    return k


def kernel(index, table):
    batch, fields = index.shape
    D = table.shape[1]
    B = batch * fields
    info = plsc.get_sparse_core_info()
    NW = info.num_cores * info.num_subcores
    b_per_w = B // NW
    idx = index.reshape(NW, b_per_w // CHUNK, CHUNK).astype(jnp.int32)
    out = _make(B, D)(idx, table)
    return out.reshape(batch, fields, D)
